# baseline (speedup 1.0000x reference)
import functools

import jax
import jax.numpy as jnp
from jax import lax
from jax.experimental import pallas as pl
from jax.experimental.pallas import tpu as pltpu
from jax.experimental.pallas import tpu_sc as plsc

_NB_OTHER = 10000
_NB_USER = 15000
_NB_ITEM = 25000
_N_NODES = _NB_OTHER + _NB_USER + _NB_ITEM
_D = 64
_HALF = 25088
_NPAD = 2 * _HALF
_TRASH = 128
_ACC_ROWS = _HALF + _TRASH
_ZROWS = _ACC_ROWS // 16

_C = 128

_mesh = plsc.VectorSubcoreMesh(core_axis_name="c", subcore_axis_name="s")
_sc_params = pltpu.CompilerParams(use_tc_tiling_on_sc=False)


_ITERS = 393
_E_PAD = (_ITERS + 1) * 16 * _C


def _make_seg_conv(n_rows_in):
    def body(x_hbm, ei_hbm, w_hbm, z_hbm, out_hbm,
             ei0, ei1, ei2, w0, w1, w2, di0, di1, di2, r0, r1, r2, acc,
             sem_in, sem_g, sem_s):
        cid = lax.axis_index("c")
        sid = lax.axis_index("s")
        rbase = cid * _HALF

        pltpu.sync_copy(z_hbm, acc.at[pl.ds(sid * _ZROWS, _ZROWS)])
        plsc.subcore_barrier()

        iota = lax.broadcasted_iota(jnp.int32, (16,), 0)
        eis = (ei0, ei1, ei2)
        ws = (w0, w1, w2)
        dis = (di0, di1, di2)
        rs = (r0, r1, r2)

        def ebase(j):
            jp = lax.rem(j + cid * (_ITERS // 2), _ITERS)
            return (sid + 16 * jp) * _C

        def issue_in(j, b):
            pltpu.async_copy(ei_hbm.at[:, pl.ds(ebase(j), _C)], eis[b], sem_in)
            pltpu.async_copy(w_hbm.at[pl.ds(ebase(j), _C)], ws[b], sem_in)

        def wait_in(b):
            pltpu.make_async_copy(ei_hbm.at[:, pl.ds(0, _C)], eis[b], sem_in).wait()
            pltpu.make_async_copy(w_hbm.at[pl.ds(0, _C)], ws[b], sem_in).wait()

        def issue_g(b):
            pltpu.async_copy(x_hbm.at[eis[b].at[0]], rs[b], sem_g)

        def wait_g(b):
            pltpu.make_async_copy(x_hbm.at[eis[b].at[0]], rs[b], sem_g).wait()

        def issue_s(b):
            pltpu.async_copy(rs[b], acc.at[dis[b]], sem_s, add=True)

        def wait_s(b):
            pltpu.make_async_copy(rs[b], acc.at[dis[b]], sem_s).wait()

        def compute(b):
            for g in range(_C // 16):
                d = eis[b][1, pl.ds(g * 16, 16)]
                local = d - rbase
                ok = (local >= 0) & (local < _HALF)
                trash = _HALF + (g % (_TRASH // 16)) * 16 + iota
                dis[b][pl.ds(g * 16, 16)] = jnp.where(ok, local, trash)
            for g in range(_C // 16):
                wvec = ws[b][pl.ds(g * 16, 16)]
                for k in range(16):
                    e = g * 16 + k
                    wsc = wvec[k]
                    for j in range(_D // 16):
                        rs[b][e, pl.ds(j * 16, 16)] = (
                            rs[b][e, pl.ds(j * 16, 16)] * wsc)

        issue_in(0, 0)
        issue_in(1, 1)
        issue_in(2, 2)
        wait_in(0)
        issue_g(0)
        wait_in(1)
        issue_g(1)

        def trip_body(t, _):
            for b in (0, 1, 2):
                i = 3 * t + b
                wait_g(b)
                wait_in((b + 2) % 3)
                if b == 0:
                    @pl.when(t > 0)
                    def _():
                        wait_s(2)
                else:
                    wait_s(b - 1)
                issue_g((b + 2) % 3)
                compute(b)
                issue_s(b)
                issue_in(i + 3, b)
            return ()

        lax.fori_loop(0, _ITERS // 3, trip_body, (), unroll=False)

        wait_s(2)
        wait_g(0)
        wait_g(1)
        wait_in(2)

        plsc.subcore_barrier()
        rpt = _HALF // 16
        pltpu.sync_copy(acc.at[pl.ds(sid * rpt, rpt)],
                        out_hbm.at[pl.ds(rbase + sid * rpt, rpt)])

    return pl.kernel(
        body,
        out_type=jax.ShapeDtypeStruct((_NPAD, _D), jnp.float32),
        mesh=_mesh,
        compiler_params=_sc_params,
        scratch_types=(
            [pltpu.VMEM((2, _C), jnp.int32)] * 3
            + [pltpu.VMEM((_C,), jnp.float32)] * 3
            + [pltpu.VMEM((_C,), jnp.int32)] * 3
            + [pltpu.VMEM((_C, _D), jnp.float32)] * 3
            + [pltpu.VMEM_SHARED((_ACC_ROWS, _D), jnp.float32)]
            + [pltpu.SemaphoreType.DMA] * 3
        ),
    )


def _make_gather2(n_rows, n_idx):
    per_tile = n_idx // 32
    n_ch = per_tile // _C

    def body(t1_hbm, t2_hbm, idx_hbm, o1_hbm, o2_hbm, idx_v, r1_v, r2_v):
        cid = lax.axis_index("c")
        sid = lax.axis_index("s")
        wid = sid * 2 + cid

        for i in range(n_ch):
            base = wid * per_tile + i * _C
            pltpu.sync_copy(idx_hbm.at[pl.ds(base, _C)], idx_v)
            pltpu.sync_copy(t1_hbm.at[idx_v], r1_v)
            pltpu.sync_copy(r1_v, o1_hbm.at[pl.ds(base, _C)])
            pltpu.sync_copy(t2_hbm.at[idx_v], r2_v)
            pltpu.sync_copy(r2_v, o2_hbm.at[pl.ds(base, _C)])

    return pl.kernel(
        body,
        out_type=(jax.ShapeDtypeStruct((n_idx, _D), jnp.float32),
                  jax.ShapeDtypeStruct((n_idx, _D), jnp.float32)),
        mesh=_mesh,
        compiler_params=_sc_params,
        scratch_types=[
            pltpu.VMEM((_C,), jnp.int32),
            pltpu.VMEM((_C, _D), jnp.float32),
            pltpu.VMEM((_C, _D), jnp.float32),
        ],
    )


_MM_BLK = 512


def _mm_body(x_ref, w_ref, b_ref, o_ref):
    o_ref[...] = jnp.dot(x_ref[...], w_ref[...],
                         preferred_element_type=jnp.float32) + b_ref[...]


def _mm(x, w, b):
    n = x.shape[0]
    return pl.pallas_call(
        _mm_body,
        grid=(n // _MM_BLK,),
        in_specs=[
            pl.BlockSpec((_MM_BLK, _D), lambda i: (i, 0)),
            pl.BlockSpec((_D, _D), lambda i: (0, 0)),
            pl.BlockSpec((1, _D), lambda i: (0, 0)),
        ],
        out_specs=pl.BlockSpec((_MM_BLK, _D), lambda i: (i, 0)),
        out_shape=jax.ShapeDtypeStruct((n, _D), jnp.float32),
    )(x, w, b.reshape(1, _D))


_B = 4096
_KNEG = 4


def _score_body(h1_ref, h2_ref, wl_ref, bl_ref, lab_ref, o_ref):
    h1 = h1_ref[...]
    h2 = h2_ref[...]
    og = jnp.dot(h2, wl_ref[...], preferred_element_type=jnp.float32) + bl_ref[...]
    u1, p1, n1 = h1[:_B], h1[_B:2 * _B], h1[2 * _B:]
    u2, p2, n2 = h2[:_B], h2[_B:2 * _B], h2[2 * _B:]
    ou, op_, on = og[:_B], og[_B:2 * _B], og[2 * _B:]

    pos = jnp.sum(u1 * p1 + u2 * p2 + ou * op_, axis=1, keepdims=True)
    n1r = n1.reshape(_KNEG, _B, _D)
    n2r = n2.reshape(_KNEG, _B, _D)
    onr = on.reshape(_KNEG, _B, _D)
    neg = jnp.sum(n1r * u1[None] + n2r * u2[None] + onr * ou[None], axis=2)

    y = lab_ref[...].astype(jnp.float32)
    y_pos = y[:_B]
    y_neg = y[_B:].reshape(_KNEG, _B)

    def bce(l, t):
        return jnp.maximum(l, 0.0) - l * t + jnp.log1p(jnp.exp(-jnp.abs(l)))

    total = jnp.sum(bce(pos, y_pos)) + jnp.sum(bce(neg, y_neg))
    o_ref[...] = (total / (_B + _KNEG * _B)).reshape(1, 1)


def _score(h1g, h2g, wl, bl, label):
    return pl.pallas_call(
        _score_body,
        out_shape=jax.ShapeDtypeStruct((1, 1), jnp.float32),
    )(h1g, h2g, wl, bl.reshape(1, _D), label.reshape(-1, 1))


_seg_conv_in = _make_seg_conv(_N_NODES)
_seg_conv_pad = _make_seg_conv(_NPAD)
_gather2 = _make_gather2(_NPAD, 24576)


def kernel(user, pos_item, neg_item, label, edge_index, edge_weight,
           other_emb, user_emb, item_emb, W1, b1, W2, b2, Wl, bl):
    all_emb = jnp.concatenate([other_emb, user_emb, item_emb], axis=0)
    n_e = edge_weight.shape[0]
    pad_ei = jnp.broadcast_to(jnp.array([[0], [_NPAD]], jnp.int32),
                              (2, _E_PAD - n_e))
    ei_pad = jnp.concatenate([edge_index, pad_ei], axis=1)
    w_pad = jnp.concatenate([edge_weight,
                             jnp.zeros((_E_PAD - n_e,), jnp.float32)])
    zseed = jnp.zeros((_ZROWS, _D), jnp.float32)

    agg1 = _seg_conv_in(all_emb, ei_pad, w_pad, zseed)
    h1 = _mm(agg1, W1, b1)
    agg2 = _seg_conv_pad(h1, ei_pad, w_pad, zseed)
    h2 = _mm(agg2, W2, b2)

    gidx = jnp.concatenate([user + _NB_OTHER,
                            pos_item + _NB_OTHER + _NB_USER,
                            neg_item + _NB_OTHER + _NB_USER])
    h1g, h2g = _gather2(h1, h2, gidx)
    loss = _score(h1g, h2g, Wl, bl, label)
    return loss[0, 0]


# speedup vs baseline: 1.1451x; 1.1451x over previous
import functools

import jax
import jax.numpy as jnp
from jax import lax
from jax.experimental import pallas as pl
from jax.experimental.pallas import tpu as pltpu
from jax.experimental.pallas import tpu_sc as plsc

_NB_OTHER = 10000
_NB_USER = 15000
_NB_ITEM = 25000
_N_NODES = _NB_OTHER + _NB_USER + _NB_ITEM
_D = 64
_HALF = 25088
_NPAD = 2 * _HALF
_TRASH = 128
_ACC_ROWS = _HALF + _TRASH
_ZROWS = _ACC_ROWS // 16

_C = 128

_mesh = plsc.VectorSubcoreMesh(core_axis_name="c", subcore_axis_name="s")
_sc_params = pltpu.CompilerParams(use_tc_tiling_on_sc=False)


_ITERS = 392
_E_PAD = (_ITERS + 2) * 16 * _C


def _make_seg_conv(n_rows_in):
    def body(x_hbm, ei_hbm, w_hbm, z_hbm, out_hbm,
             ei0, ei1, w0, w1, di0, di1, r0, r1, acc,
             sem_in, sem_g, sem_s):
        cid = lax.axis_index("c")
        sid = lax.axis_index("s")
        rbase = cid * _HALF

        pltpu.sync_copy(z_hbm, acc.at[pl.ds(sid * _ZROWS, _ZROWS)])
        plsc.subcore_barrier()

        iota = lax.broadcasted_iota(jnp.int32, (16,), 0)
        eis = (ei0, ei1)
        ws = (w0, w1)
        dis = (di0, di1)
        rs = (r0, r1)

        def ebase(j):
            jp = lax.rem(j + cid * (_ITERS // 2), _ITERS)
            return (sid + 16 * jp) * _C

        def issue_in(j, b):
            pltpu.async_copy(ei_hbm.at[:, pl.ds(ebase(j), _C)], eis[b], sem_in)
            pltpu.async_copy(w_hbm.at[pl.ds(ebase(j), _C)], ws[b], sem_in)

        def wait_in(b):
            pltpu.make_async_copy(ei_hbm.at[:, pl.ds(0, _C)], eis[b], sem_in).wait()
            pltpu.make_async_copy(w_hbm.at[pl.ds(0, _C)], ws[b], sem_in).wait()

        def issue_g(b):
            pltpu.async_copy(x_hbm.at[eis[b].at[0]], rs[b], sem_g)

        def wait_g(b):
            pltpu.make_async_copy(x_hbm.at[eis[b].at[0]], rs[b], sem_g).wait()

        def issue_s(b):
            pltpu.async_copy(rs[b], acc.at[dis[b]], sem_s, add=True)

        def wait_s(b):
            pltpu.make_async_copy(rs[b], acc.at[dis[b]], sem_s).wait()

        def compute(b):
            for g in range(_C // 16):
                d = eis[b][1, pl.ds(g * 16, 16)]
                local = d - rbase
                ok = (local >= 0) & (local < _HALF)
                trash = _HALF + (g % (_TRASH // 16)) * 16 + iota
                dis[b][pl.ds(g * 16, 16)] = jnp.where(ok, local, trash)
            for g in range(_C // 16):
                wvec = ws[b][pl.ds(g * 16, 16)]
                for k in range(16):
                    e = g * 16 + k
                    wsc = wvec[k]
                    for j in range(_D // 16):
                        rs[b][e, pl.ds(j * 16, 16)] = (
                            rs[b][e, pl.ds(j * 16, 16)] * wsc)

        issue_in(0, 0)
        issue_in(1, 1)
        wait_in(0)
        issue_g(0)

        def pair_body(t, _):
            for b in (0, 1):
                i = 2 * t + b
                wait_g(b)
                wait_in(1 - b)
                if b == 0:
                    @pl.when(t > 0)
                    def _():
                        wait_s(1)
                else:
                    wait_s(0)
                issue_g(1 - b)
                compute(b)
                issue_s(b)
                issue_in(i + 2, b)
            return ()

        lax.fori_loop(0, _ITERS // 2, pair_body, (), unroll=False)

        wait_s(1)
        wait_g(0)
        wait_in(0)

        plsc.subcore_barrier()
        rpt = _HALF // 16
        pltpu.sync_copy(acc.at[pl.ds(sid * rpt, rpt)],
                        out_hbm.at[pl.ds(rbase + sid * rpt, rpt)])

    return pl.kernel(
        body,
        out_type=jax.ShapeDtypeStruct((_NPAD, _D), jnp.float32),
        mesh=_mesh,
        compiler_params=_sc_params,
        scratch_types=(
            [pltpu.VMEM((2, _C), jnp.int32)] * 2
            + [pltpu.VMEM((_C,), jnp.float32)] * 2
            + [pltpu.VMEM((_C,), jnp.int32)] * 2
            + [pltpu.VMEM((_C, _D), jnp.float32)] * 2
            + [pltpu.VMEM_SHARED((_ACC_ROWS, _D), jnp.float32)]
            + [pltpu.SemaphoreType.DMA] * 3
        ),
    )


def _make_gather2(n_rows, n_idx):
    per_tile = n_idx // 32
    n_ch = per_tile // _C

    def body(t1_hbm, t2_hbm, idx_hbm, o1_hbm, o2_hbm, idx_v, r1_v, r2_v):
        cid = lax.axis_index("c")
        sid = lax.axis_index("s")
        wid = sid * 2 + cid

        for i in range(n_ch):
            base = wid * per_tile + i * _C
            pltpu.sync_copy(idx_hbm.at[pl.ds(base, _C)], idx_v)
            pltpu.sync_copy(t1_hbm.at[idx_v], r1_v)
            pltpu.sync_copy(r1_v, o1_hbm.at[pl.ds(base, _C)])
            pltpu.sync_copy(t2_hbm.at[idx_v], r2_v)
            pltpu.sync_copy(r2_v, o2_hbm.at[pl.ds(base, _C)])

    return pl.kernel(
        body,
        out_type=(jax.ShapeDtypeStruct((n_idx, _D), jnp.float32),
                  jax.ShapeDtypeStruct((n_idx, _D), jnp.float32)),
        mesh=_mesh,
        compiler_params=_sc_params,
        scratch_types=[
            pltpu.VMEM((_C,), jnp.int32),
            pltpu.VMEM((_C, _D), jnp.float32),
            pltpu.VMEM((_C, _D), jnp.float32),
        ],
    )


_MM_BLK = 512


def _mm_body(x_ref, w_ref, b_ref, o_ref):
    o_ref[...] = jnp.dot(x_ref[...], w_ref[...],
                         preferred_element_type=jnp.float32) + b_ref[...]


def _mm(x, w, b):
    n = x.shape[0]
    return pl.pallas_call(
        _mm_body,
        grid=(n // _MM_BLK,),
        in_specs=[
            pl.BlockSpec((_MM_BLK, _D), lambda i: (i, 0)),
            pl.BlockSpec((_D, _D), lambda i: (0, 0)),
            pl.BlockSpec((1, _D), lambda i: (0, 0)),
        ],
        out_specs=pl.BlockSpec((_MM_BLK, _D), lambda i: (i, 0)),
        out_shape=jax.ShapeDtypeStruct((n, _D), jnp.float32),
    )(x, w, b.reshape(1, _D))


_B = 4096
_KNEG = 4


def _score_body(h1_ref, h2_ref, wl_ref, bl_ref, lab_ref, o_ref):
    h1 = h1_ref[...]
    h2 = h2_ref[...]
    og = jnp.dot(h2, wl_ref[...], preferred_element_type=jnp.float32) + bl_ref[...]
    u1, p1, n1 = h1[:_B], h1[_B:2 * _B], h1[2 * _B:]
    u2, p2, n2 = h2[:_B], h2[_B:2 * _B], h2[2 * _B:]
    ou, op_, on = og[:_B], og[_B:2 * _B], og[2 * _B:]

    pos = jnp.sum(u1 * p1 + u2 * p2 + ou * op_, axis=1, keepdims=True)
    n1r = n1.reshape(_KNEG, _B, _D)
    n2r = n2.reshape(_KNEG, _B, _D)
    onr = on.reshape(_KNEG, _B, _D)
    neg = jnp.sum(n1r * u1[None] + n2r * u2[None] + onr * ou[None], axis=2)

    y = lab_ref[...].astype(jnp.float32)
    y_pos = y[:_B]
    y_neg = y[_B:].reshape(_KNEG, _B)

    def bce(l, t):
        return jnp.maximum(l, 0.0) - l * t + jnp.log1p(jnp.exp(-jnp.abs(l)))

    total = jnp.sum(bce(pos, y_pos)) + jnp.sum(bce(neg, y_neg))
    o_ref[...] = (total / (_B + _KNEG * _B)).reshape(1, 1)


def _score(h1g, h2g, wl, bl, label):
    return pl.pallas_call(
        _score_body,
        out_shape=jax.ShapeDtypeStruct((1, 1), jnp.float32),
    )(h1g, h2g, wl, bl.reshape(1, _D), label.reshape(-1, 1))


_seg_conv_in = _make_seg_conv(_N_NODES)
_seg_conv_pad = _make_seg_conv(_NPAD)
_gather2 = _make_gather2(_NPAD, 24576)


def kernel(user, pos_item, neg_item, label, edge_index, edge_weight,
           other_emb, user_emb, item_emb, W1, b1, W2, b2, Wl, bl):
    all_emb = jnp.concatenate([other_emb, user_emb, item_emb], axis=0)
    n_e = edge_weight.shape[0]
    pad_ei = jnp.broadcast_to(jnp.array([[0], [_NPAD]], jnp.int32),
                              (2, _E_PAD - n_e))
    ei_pad = jnp.concatenate([edge_index, pad_ei], axis=1)
    w_pad = jnp.concatenate([edge_weight,
                             jnp.zeros((_E_PAD - n_e,), jnp.float32)])
    zseed = jnp.zeros((_ZROWS, _D), jnp.float32)

    agg1 = _seg_conv_in(all_emb, ei_pad, w_pad, zseed)
    h1 = _mm(agg1, W1, b1)
    agg2 = _seg_conv_pad(h1, ei_pad, w_pad, zseed)
    h2 = _mm(agg2, W2, b2)

    gidx = jnp.concatenate([user + _NB_OTHER,
                            pos_item + _NB_OTHER + _NB_USER,
                            neg_item + _NB_OTHER + _NB_USER])
    h1g, h2g = _gather2(h1, h2, gidx)
    loss = _score(h1g, h2g, Wl, bl, label)
    return loss[0, 0]
